# baseline (speedup 1.0000x reference)
import functools

import jax
import jax.numpy as jnp
from jax import lax
from jax.experimental import pallas as pl
from jax.experimental.pallas import tpu as pltpu

_VOCAB = 1000000
_EMBED = 32
_BATCH = 4096

_GB = 128
_GSTEPS = _BATCH // _GB


def _gather_body(sidx_ref, et_ref, o_ref, bufs_ref, sem):
  i = pl.program_id(0)
  base = i * _GB
  copies = []
  for k in range(_GB):
    idx = sidx_ref[base + k]
    start = pl.multiple_of((idx // 128) * 128, 128)
    cp = pltpu.make_async_copy(
        et_ref.at[:, pl.ds(start, 128)],
        bufs_ref.at[k],
        sem)
    cp.start()
    copies.append(cp)
  for cp in copies:
    cp.wait()
  kiota = lax.broadcasted_iota(jnp.int32, (_EMBED, 128), 1)
  acc = jnp.zeros((_EMBED, 128), jnp.float32)
  for k in range(_GB):
    idx = sidx_ref[base + k]
    lane = lax.rem(idx, 128)
    shift = lax.rem(k - lane + 128, 128)
    rolled = pltpu.roll(bufs_ref[k], shift, axis=1)
    acc = jnp.where(kiota == k, rolled, acc)
  o_ref[...] = acc


_gather_t = pl.pallas_call(
    _gather_body,
    grid_spec=pltpu.PrefetchScalarGridSpec(
        num_scalar_prefetch=1,
        grid=(_GSTEPS,),
        in_specs=[pl.BlockSpec(memory_space=pltpu.MemorySpace.HBM)],
        out_specs=pl.BlockSpec((_EMBED, _GB), lambda i, s: (0, i)),
        scratch_shapes=[
            pltpu.VMEM((_GB, _EMBED, 128), jnp.float32),
            pltpu.SemaphoreType.DMA,
        ],
    ),
    out_shape=jax.ShapeDtypeStruct((_EMBED, _BATCH), jnp.float32),
)

_BM = 256


def _matmul_body(a_ref, b_ref, o_ref):
  o_ref[...] = lax.dot_general(
      a_ref[...], b_ref[...],
      dimension_numbers=(((0,), (0,)), ((), ())),
      preferred_element_type=jnp.float32)


_matmul = pl.pallas_call(
    _matmul_body,
    grid=(_BATCH // _BM,),
    in_specs=[
        pl.BlockSpec((_EMBED, _BM), lambda i: (0, i)),
        pl.BlockSpec((_EMBED, _BATCH), lambda i: (0, 0)),
    ],
    out_specs=pl.BlockSpec((_BM, _BATCH), lambda i: (i, 0)),
    out_shape=jax.ShapeDtypeStruct((_BATCH, _BATCH), jnp.float32),
)


@jax.jit
def kernel(target, context, embeddings):
  et = embeddings.T
  rows_t = _gather_t(target.astype(jnp.int32), et)
  rows_c = _gather_t(context.astype(jnp.int32), et)
  return _matmul(rows_t, rows_c)


# speedup vs baseline: 1.4772x; 1.4772x over previous
import jax
import jax.numpy as jnp
from jax import lax
from jax.experimental import pallas as pl
from jax.experimental.pallas import tpu as pltpu

_VOCAB = 1000000
_EMBED = 32
_BATCH = 4096

_GB = 128
_GSTEPS = _BATCH // _GB


def _issue(sidx_ref, et_ref, bufs_ref, sems_ref, step, slot):
  base = step * _GB
  copies = []
  for k in range(_GB):
    idx = sidx_ref[base + k]
    start = pl.multiple_of((idx // 128) * 128, 128)
    copies.append(pltpu.make_async_copy(
        et_ref.at[:, pl.ds(start, 128)],
        bufs_ref.at[slot, k],
        sems_ref.at[slot]))
  return copies


def _gather_body(sidx_ref, et_ref, o_ref, bufs_ref, sems_ref):
  i = pl.program_id(0)

  @pl.when(i == 0)
  def _prime():
    for cp in _issue(sidx_ref, et_ref, bufs_ref, sems_ref, 0, 0):
      cp.start()

  @pl.when(i + 1 < _GSTEPS)
  def _ahead():
    for cp in _issue(sidx_ref, et_ref, bufs_ref, sems_ref,
                     (i + 1) % _GSTEPS, (i + 1) % 2):
      cp.start()

  for cp in _issue(sidx_ref, et_ref, bufs_ref, sems_ref, i % _GSTEPS, i % 2):
    cp.wait()

  base = i * _GB
  slot = i % 2
  kiota = lax.broadcasted_iota(jnp.int32, (_EMBED, 128), 1)
  acc = jnp.zeros((_EMBED, 128), jnp.float32)
  for k in range(_GB):
    idx = sidx_ref[base + k]
    lane = lax.rem(idx, 128)
    shift = lax.rem(k - lane + 128, 128)
    rolled = pltpu.roll(bufs_ref[slot, k], shift, axis=1)
    acc = jnp.where(kiota == k, rolled, acc)
  o_ref[...] = acc


_gather_t = pl.pallas_call(
    _gather_body,
    grid_spec=pltpu.PrefetchScalarGridSpec(
        num_scalar_prefetch=1,
        grid=(_GSTEPS,),
        in_specs=[pl.BlockSpec(memory_space=pltpu.MemorySpace.HBM)],
        out_specs=pl.BlockSpec((_EMBED, _GB), lambda i, s: (0, i)),
        scratch_shapes=[
            pltpu.VMEM((2, _GB, _EMBED, 128), jnp.float32),
            pltpu.SemaphoreType.DMA((2,)),
        ],
    ),
    out_shape=jax.ShapeDtypeStruct((_EMBED, _BATCH), jnp.float32),
)

_BM = 256


def _matmul_body(a_ref, b_ref, o_ref):
  o_ref[...] = lax.dot_general(
      a_ref[...], b_ref[...],
      dimension_numbers=(((0,), (0,)), ((), ())),
      preferred_element_type=jnp.float32)


_matmul = pl.pallas_call(
    _matmul_body,
    grid=(_BATCH // _BM,),
    in_specs=[
        pl.BlockSpec((_EMBED, _BM), lambda i: (0, i)),
        pl.BlockSpec((_EMBED, _BATCH), lambda i: (0, 0)),
    ],
    out_specs=pl.BlockSpec((_BM, _BATCH), lambda i: (i, 0)),
    out_shape=jax.ShapeDtypeStruct((_BATCH, _BATCH), jnp.float32),
)


@jax.jit
def kernel(target, context, embeddings):
  et = embeddings.T
  rows_t = _gather_t(target.astype(jnp.int32), et)
  rows_c = _gather_t(context.astype(jnp.int32), et)
  return _matmul(rows_t, rows_c)


# speedup vs baseline: 1.4959x; 1.0127x over previous
import jax
import jax.numpy as jnp
from jax import lax
from jax.experimental import pallas as pl
from jax.experimental.pallas import tpu as pltpu

_VOCAB = 1000000
_EMBED = 32
_BATCH = 4096

_GB = 128
_GSTEPS = _BATCH // _GB


def _issue(sidx_ref, et_ref, bufs_ref, sems_ref, step, slot):
  base = step * _GB
  copies = []
  for k in range(_GB):
    idx = sidx_ref[base + k]
    start = pl.multiple_of((idx // 128) * 128, 128)
    copies.append(pltpu.make_async_copy(
        et_ref.at[:, pl.ds(start, 128)],
        bufs_ref.at[slot, k],
        sems_ref.at[slot]))
  return copies


def _gather_body(sidx_ref, et_ref, o_ref, bufs_ref, sems_ref):
  i = pl.program_id(0)

  @pl.when(i == 0)
  def _prime():
    for cp in _issue(sidx_ref, et_ref, bufs_ref, sems_ref, 0, 0):
      cp.start()

  @pl.when(i + 1 < _GSTEPS)
  def _ahead():
    for cp in _issue(sidx_ref, et_ref, bufs_ref, sems_ref,
                     (i + 1) % _GSTEPS, (i + 1) % 2):
      cp.start()

  for cp in _issue(sidx_ref, et_ref, bufs_ref, sems_ref, i % _GSTEPS, i % 2):
    cp.wait()

  base = i * _GB
  slot = i % 2
  kiota = lax.broadcasted_iota(jnp.int32, (_EMBED, 128), 1)
  acc = jnp.zeros((_EMBED, 128), jnp.float32)
  for k in range(_GB):
    idx = sidx_ref[base + k]
    lane = lax.rem(idx, 128)
    shift = lax.rem(k - lane + 128, 128)
    rolled = pltpu.roll(bufs_ref[slot, k], shift, axis=1)
    acc = jnp.where(kiota == k, rolled, acc)
  o_ref[...] = acc


_gather_t = pl.pallas_call(
    _gather_body,
    grid_spec=pltpu.PrefetchScalarGridSpec(
        num_scalar_prefetch=1,
        grid=(_GSTEPS,),
        in_specs=[pl.BlockSpec(memory_space=pltpu.MemorySpace.HBM)],
        out_specs=pl.BlockSpec((_EMBED, _GB), lambda i, s: (0, i)),
        scratch_shapes=[
            pltpu.VMEM((2, _GB, _EMBED, 128), jnp.float32),
            pltpu.SemaphoreType.DMA((2,)),
        ],
    ),
    out_shape=jax.ShapeDtypeStruct((_EMBED, _BATCH), jnp.float32),
)

_BM = 512


def _matmul_body(a_ref, b_ref, o_ref):
  o_ref[...] = lax.dot_general(
      a_ref[...], b_ref[...],
      dimension_numbers=(((0,), (0,)), ((), ())),
      preferred_element_type=jnp.float32)


_matmul = pl.pallas_call(
    _matmul_body,
    grid=(_BATCH // _BM,),
    in_specs=[
        pl.BlockSpec((_EMBED, _BM), lambda i: (0, i)),
        pl.BlockSpec((_EMBED, _BATCH), lambda i: (0, 0)),
    ],
    out_specs=pl.BlockSpec((_BM, _BATCH), lambda i: (i, 0)),
    out_shape=jax.ShapeDtypeStruct((_BATCH, _BATCH), jnp.float32),
)


@jax.jit
def kernel(target, context, embeddings):
  et = embeddings.T
  rows_t = _gather_t(target.astype(jnp.int32), et)
  rows_c = _gather_t(context.astype(jnp.int32), et)
  return _matmul(rows_t, rows_c)
